# baseline (speedup 1.0000x reference)
import functools

import jax
import jax.numpy as jnp
from jax import lax
from jax.experimental import pallas as pl
from jax.experimental.pallas import tpu as pltpu
from jax.experimental.pallas import tpu_sc as plsc

N_LAYERS = 12
D_SAE = 16384
D_MODEL = 768
NNZ = 65536

LANES = 16
NUM_WORKERS = 32
PER_W = NNZ // NUM_WORKERS
CHUNK = 64
N_CHUNKS = PER_W // CHUNK
COLS = D_MODEL // LANES


def _body(layer_hbm, feat_hbm, acts_hbm, table_hbm, out_hbm,
          idx_v, feat_v, acts_v, rows0, rows1, gsem0, gsem1, osem0, osem1):
    c = lax.axis_index("c")
    s = lax.axis_index("s")
    wid = s * 2 + c
    base = wid * PER_W

    rows = (rows0, rows1)
    gsem = (gsem0, gsem1)
    osem = (osem0, osem1)

    pltpu.sync_copy(acts_hbm.at[pl.ds(base, PER_W)], acts_v.at[pl.ds(0, PER_W)])
    pltpu.sync_copy(layer_hbm.at[pl.ds(base, PER_W)], idx_v)
    pltpu.sync_copy(feat_hbm.at[pl.ds(base, PER_W)], feat_v)

    def idx_body(i, carry):
        off = i * LANES
        l = idx_v[pl.ds(off, LANES)]
        f = feat_v[pl.ds(off, LANES)]
        idx_v[pl.ds(off, LANES)] = (l << 14) + f
        return carry

    lax.fori_loop(0, PER_W // LANES, idx_body, 0, unroll=4)

    def gather(g, b):
        return pltpu.async_copy(
            table_hbm.at[idx_v.at[pl.ds(g * CHUNK, CHUNK)]], rows[b], gsem[b])

    HALF = CHUNK // 2

    def out_copy_half(g, b, h):
        return pltpu.async_copy(
            rows[b].at[pl.ds(h * HALF, HALF)],
            out_hbm.at[pl.ds(base + g * CHUNK + h * HALF, HALF)], osem[b])

    def wait_gather(g, b):
        pltpu.make_async_copy(
            table_hbm.at[idx_v.at[pl.ds(g * CHUNK, CHUNK)]], rows[b], gsem[b]
        ).wait()

    def wait_out(g, b):
        for h in range(2):
            pltpu.make_async_copy(
                rows[b].at[pl.ds(h * HALF, HALF)],
                out_hbm.at[pl.ds(base + g * CHUNK + h * HALF, HALF)], osem[b]
            ).wait()

    def scale_half(g, b, h):
        def row_body(r, carry2):
            m = g * CHUNK + h * HALF + r
            a = jnp.full((LANES,), acts_v[pl.ds(m, LANES)][0])
            for cc in range(COLS):
                sl = pl.ds(cc * LANES, LANES)
                rows[b][h * HALF + r, sl] = rows[b][h * HALF + r, sl] * a
            return carry2

        lax.fori_loop(0, HALF, row_body, 0)

    gather(0, 0)

    def pair_body(g2, carry):
        for b in range(2):
            g = g2 * 2 + b
            wait_gather(g, b)

            @pl.when(g + 1 < N_CHUNKS)
            def _start_next():
                @pl.when(g >= 1)
                def _drain_prev_out():
                    wait_out(g - 1, 1 - b)

                gather(g + 1, 1 - b)

            scale_half(g, b, 0)
            out_copy_half(g, b, 0)
            scale_half(g, b, 1)
            out_copy_half(g, b, 1)
        return carry

    lax.fori_loop(0, N_CHUNKS // 2, pair_body, 0)
    wait_out(N_CHUNKS - 2, 0)
    wait_out(N_CHUNKS - 1, 1)


def kernel(layer_idx, pos_idx, feat_idx, activations, W_D):
    table = W_D.reshape(N_LAYERS * D_SAE, D_MODEL)
    mesh = plsc.VectorSubcoreMesh(core_axis_name="c", subcore_axis_name="s")
    scaled = pl.kernel(
        _body,
        mesh=mesh,
        out_type=jax.ShapeDtypeStruct((NNZ, D_MODEL), jnp.float32),
        scratch_types=[
            pltpu.VMEM((PER_W,), jnp.int32),
            pltpu.VMEM((PER_W,), jnp.int32),
            pltpu.VMEM((PER_W + LANES,), jnp.float32),
            pltpu.VMEM((CHUNK, D_MODEL), jnp.float32),
            pltpu.VMEM((CHUNK, D_MODEL), jnp.float32),
            pltpu.SemaphoreType.DMA,
            pltpu.SemaphoreType.DMA,
            pltpu.SemaphoreType.DMA,
            pltpu.SemaphoreType.DMA,
        ],
    )(layer_idx.astype(jnp.int32), feat_idx.astype(jnp.int32),
      activations, table)
    encoder_mapping = jnp.arange(NNZ, dtype=jnp.int32)
    return (pos_idx, layer_idx, feat_idx, scaled, encoder_mapping)


# speedup vs baseline: 1.0160x; 1.0160x over previous
import jax
import jax.numpy as jnp
from jax import lax
from jax.experimental import pallas as pl
from jax.experimental.pallas import tpu as pltpu
from jax.experimental.pallas import tpu_sc as plsc

N_LAYERS = 12
D_SAE = 16384
D_MODEL = 768
NNZ = 65536

LANES = 16
NUM_WORKERS = 32
PER_W = NNZ // NUM_WORKERS
CHUNK = 64
N_CHUNKS = PER_W // CHUNK
COLS = D_MODEL // LANES


def _body(layer_hbm, feat_hbm, acts_hbm, table_hbm, out_hbm,
          idx_v, feat_v, acts_v, rows0, rows1, gsem0, gsem1, osem0, osem1):
    c = lax.axis_index("c")
    s = lax.axis_index("s")
    wid = s * 2 + c
    base = wid * PER_W

    rows = (rows0, rows1)
    gsem = (gsem0, gsem1)
    osem = (osem0, osem1)

    pltpu.sync_copy(acts_hbm.at[pl.ds(base, PER_W)], acts_v.at[pl.ds(0, PER_W)])
    pltpu.sync_copy(layer_hbm.at[pl.ds(base, PER_W)], idx_v)
    pltpu.sync_copy(feat_hbm.at[pl.ds(base, PER_W)], feat_v)

    def idx_body(i, carry):
        off = i * LANES
        l = idx_v[pl.ds(off, LANES)]
        f = feat_v[pl.ds(off, LANES)]
        idx_v[pl.ds(off, LANES)] = (l << 14) + f
        return carry

    lax.fori_loop(0, PER_W // LANES, idx_body, 0, unroll=4)

    def gather(g, b):
        return pltpu.async_copy(
            table_hbm.at[idx_v.at[pl.ds(g * CHUNK, CHUNK)]], rows[b], gsem[b])

    def out_copy(g, b):
        return pltpu.async_copy(
            rows[b], out_hbm.at[pl.ds(base + g * CHUNK, CHUNK)], osem[b])

    def wait_gather(g, b):
        pltpu.make_async_copy(
            table_hbm.at[idx_v.at[pl.ds(g * CHUNK, CHUNK)]], rows[b], gsem[b]
        ).wait()

    def wait_out(g, b):
        pltpu.make_async_copy(
            rows[b], out_hbm.at[pl.ds(base + g * CHUNK, CHUNK)], osem[b]
        ).wait()

    def scale(g, b):
        def row_body(r, carry2):
            m = g * CHUNK + r
            a = jnp.full((LANES,), acts_v[pl.ds(m, LANES)][0])
            for cc in range(COLS):
                sl = pl.ds(cc * LANES, LANES)
                rows[b][r, sl] = rows[b][r, sl] * a
            return carry2

        lax.fori_loop(0, CHUNK, row_body, 0)

    gather(0, 0)

    def pair_body(g2, carry):
        for b in range(2):
            g = g2 * 2 + b
            wait_gather(g, b)

            @pl.when(g + 1 < N_CHUNKS)
            def _start_next():
                @pl.when(g >= 1)
                def _drain_prev_out():
                    wait_out(g - 1, 1 - b)

                gather(g + 1, 1 - b)

            scale(g, b)
            out_copy(g, b)
        return carry

    lax.fori_loop(0, N_CHUNKS // 2, pair_body, 0)
    wait_out(N_CHUNKS - 2, 0)
    wait_out(N_CHUNKS - 1, 1)


def kernel(layer_idx, pos_idx, feat_idx, activations, W_D):
    table = W_D.reshape(N_LAYERS * D_SAE, D_MODEL)
    mesh = plsc.VectorSubcoreMesh(core_axis_name="c", subcore_axis_name="s")
    scaled = pl.kernel(
        _body,
        mesh=mesh,
        out_type=jax.ShapeDtypeStruct((NNZ, D_MODEL), jnp.float32),
        scratch_types=[
            pltpu.VMEM((PER_W,), jnp.int32),
            pltpu.VMEM((PER_W,), jnp.int32),
            pltpu.VMEM((PER_W + LANES,), jnp.float32),
            pltpu.VMEM((CHUNK, D_MODEL), jnp.float32),
            pltpu.VMEM((CHUNK, D_MODEL), jnp.float32),
            pltpu.SemaphoreType.DMA,
            pltpu.SemaphoreType.DMA,
            pltpu.SemaphoreType.DMA,
            pltpu.SemaphoreType.DMA,
        ],
    )(layer_idx.astype(jnp.int32), feat_idx.astype(jnp.int32),
      activations, table)
    encoder_mapping = jnp.arange(NNZ, dtype=jnp.int32)
    return (pos_idx, layer_idx, feat_idx, scaled, encoder_mapping)
